# baseline (speedup 1.0000x reference)
import functools

import jax
import jax.numpy as jnp
from jax import lax
from jax.experimental import pallas as pl
from jax.experimental.pallas import tpu as pltpu
from jax.experimental.pallas import tpu_sc as plsc

N = 100000
D_FEAT = 128
EMB = 128
NF_EMB = 64
B = 16384
OUT_D = EMB + NF_EMB

NUM_CORES = 2
NUM_SUBCORES = 16
NUM_WORKERS = NUM_CORES * NUM_SUBCORES
B_PER_W = B // NUM_WORKERS

BN = 8192
BK = 2048


def _sc_gather2(emb_weight, node_feats, batch):
    mesh = plsc.VectorSubcoreMesh(core_axis_name="c", subcore_axis_name="s")

    @functools.partial(
        pl.kernel,
        mesh=mesh,
        out_type=(
            jax.ShapeDtypeStruct((B, EMB), jnp.float32),
            jax.ShapeDtypeStruct((B, D_FEAT), jnp.float32),
        ),
        scratch_types=[
            pltpu.VMEM((B_PER_W,), jnp.int32),
            pltpu.VMEM((B_PER_W, EMB), jnp.float32),
            pltpu.SemaphoreType.DMA,
        ],
    )
    def k(emb_hbm, nf_hbm, idx_hbm, emb_out, nf_out, idx_v, rows_v, sem):
        wid = lax.axis_index("s") * NUM_CORES + lax.axis_index("c")
        base = wid * B_PER_W
        pltpu.sync_copy(idx_hbm.at[pl.ds(base, B_PER_W)], idx_v)
        pltpu.async_copy(emb_hbm.at[idx_v], rows_v, sem).wait()
        pltpu.sync_copy(rows_v, emb_out.at[pl.ds(base, B_PER_W)])
        pltpu.async_copy(nf_hbm.at[idx_v], rows_v, sem).wait()
        pltpu.sync_copy(rows_v, nf_out.at[pl.ds(base, B_PER_W)])

    return k(emb_weight, node_feats, batch)


def _tc_fuse_kernel(emb_ref, nf_ref, w_ref, b_ref, out_ref):
    k = pl.program_id(1)
    zT = jax.lax.dot_general(
        w_ref[...], nf_ref[...],
        dimension_numbers=(((0,), (1,)), ((), ())),
        preferred_element_type=jnp.float32,
    )
    out_ref[:EMB, pl.ds(k * BK, BK)] = emb_ref[...].T
    out_ref[EMB:, pl.ds(k * BK, BK)] = zT + b_ref[...]


def _tc_fuse(emb_rows, nf_rows, lin_W, lin_b):
    nk = BN // BK
    return pl.pallas_call(
        _tc_fuse_kernel,
        grid=(B // BN, nk),
        in_specs=[
            pl.BlockSpec((BK, EMB), lambda j, k: (j * (BN // BK) + k, 0)),
            pl.BlockSpec((BK, D_FEAT), lambda j, k: (j * (BN // BK) + k, 0)),
            pl.BlockSpec((D_FEAT, NF_EMB), lambda j, k: (0, 0)),
            pl.BlockSpec((NF_EMB, 1), lambda j, k: (0, 0)),
        ],
        out_specs=pl.BlockSpec((OUT_D, BN), lambda j, k: (0, j)),
        out_shape=jax.ShapeDtypeStruct((OUT_D, B), jnp.float32),
    )(emb_rows, nf_rows, lin_W, lin_b)


def kernel(node_feats, emb_weight, lin_W, lin_b, batch):
    emb_rows, nf_rows = _sc_gather2(emb_weight, node_feats, batch)
    return _tc_fuse(emb_rows, nf_rows, lin_W, lin_b.reshape(NF_EMB, 1)).T


# speedup vs baseline: 1.0864x; 1.0864x over previous
import functools

import jax
import jax.numpy as jnp
from jax import lax
from jax.experimental import pallas as pl
from jax.experimental.pallas import tpu as pltpu
from jax.experimental.pallas import tpu_sc as plsc

N = 100000
D_FEAT = 128
EMB = 128
NF_EMB = 64
B = 16384
OUT_D = EMB + NF_EMB

NUM_CORES = 2
NUM_SUBCORES = 16
NUM_WORKERS = NUM_CORES * NUM_SUBCORES
B_PER_W = B // NUM_WORKERS

BN = 8192


def _sc_gather2(emb_weight, node_feats, batch):
    mesh = plsc.VectorSubcoreMesh(core_axis_name="c", subcore_axis_name="s")

    @functools.partial(
        pl.kernel,
        mesh=mesh,
        out_type=(
            jax.ShapeDtypeStruct((B, EMB), jnp.float32),
            jax.ShapeDtypeStruct((B, D_FEAT), jnp.float32),
        ),
        scratch_types=[
            pltpu.VMEM((B_PER_W,), jnp.int32),
            pltpu.VMEM((B_PER_W, EMB), jnp.float32),
            pltpu.SemaphoreType.DMA,
        ],
    )
    def k(emb_hbm, nf_hbm, idx_hbm, emb_out, nf_out, idx_v, rows_v, sem):
        wid = lax.axis_index("s") * NUM_CORES + lax.axis_index("c")
        base = wid * B_PER_W
        pltpu.sync_copy(idx_hbm.at[pl.ds(base, B_PER_W)], idx_v)
        pltpu.async_copy(emb_hbm.at[idx_v], rows_v, sem).wait()
        pltpu.sync_copy(rows_v, emb_out.at[pl.ds(base, B_PER_W)])
        pltpu.async_copy(nf_hbm.at[idx_v], rows_v, sem).wait()
        pltpu.sync_copy(rows_v, nf_out.at[pl.ds(base, B_PER_W)])

    return k(emb_weight, node_feats, batch)


def _tc_fuse_kernel(emb_ref, nf_ref, w_ref, b_ref, out_ref):
    zT = jax.lax.dot_general(
        w_ref[...], nf_ref[...],
        dimension_numbers=(((0,), (1,)), ((), ())),
        preferred_element_type=jnp.float32,
    )
    out_ref[:EMB, :] = emb_ref[...].T
    out_ref[EMB:, :] = zT + b_ref[...]


def _tc_fuse(emb_rows, nf_rows, lin_W, lin_b):
    return pl.pallas_call(
        _tc_fuse_kernel,
        grid=(B // BN,),
        in_specs=[
            pl.BlockSpec((BN, EMB), lambda i: (i, 0)),
            pl.BlockSpec((BN, D_FEAT), lambda i: (i, 0)),
            pl.BlockSpec((D_FEAT, NF_EMB), lambda i: (0, 0)),
            pl.BlockSpec((NF_EMB, 1), lambda i: (0, 0)),
        ],
        out_specs=pl.BlockSpec((OUT_D, BN), lambda i: (0, i)),
        out_shape=jax.ShapeDtypeStruct((OUT_D, B), jnp.float32),
    )(emb_rows, nf_rows, lin_W, lin_b)


def kernel(node_feats, emb_weight, lin_W, lin_b, batch):
    emb_rows, nf_rows = _sc_gather2(emb_weight, node_feats, batch)
    return _tc_fuse(emb_rows, nf_rows, lin_W, lin_b.reshape(NF_EMB, 1)).T
